# baseline (speedup 1.0000x reference)
import functools

import jax
import jax.numpy as jnp
from jax.experimental import pallas as pl

_K = 32
_BLOCK_ROWS = 256


def _count_tree(msk01):
    h = msk01.shape[0]
    while h > 32:
        q = h // 4
        msk01 = ((msk01[:q, :] + msk01[q:2 * q, :])
                 + (msk01[2 * q:3 * q, :] + msk01[3 * q:, :]))
        h = q
    return jnp.sum(msk01.astype(jnp.int32), axis=0, keepdims=True)


def _sae_block_kernel(xt_ref, wenc_ref, b_enc_ref, wdec_ref, b_dec_ref,
                      out_ref):
    sae_in_t = xt_ref[...] - b_dec_ref[...]
    pre = jnp.dot(wenc_ref[...], sae_in_t,
                  preferred_element_type=jnp.float32)
    acts = jnp.maximum(pre + b_enc_ref[...], 0.0)
    bits = jax.lax.bitcast_convert_type(acts, jnp.int32)

    hi16 = (bits >> 16).astype(jnp.int16)
    lo16 = (bits ^ 0x8000).astype(jnp.int16)

    cols = acts.shape[1]

    def body1(i, p):
        maskb = (jnp.int32(1) << (14 - i)).astype(jnp.int16)
        probe = p | maskb
        cnt = _count_tree((hi16 >= probe).astype(jnp.int16))
        s = ((cnt - _K) >> 31).astype(jnp.int16)
        return p | (maskb & ~s)

    T = jax.lax.fori_loop(0, 15, body1, jnp.zeros((1, cols), jnp.int16))

    c_hi = _count_tree((hi16 > T).astype(jnp.int16))
    r = _K - c_hi

    w = jnp.where(hi16 == T, lo16, jnp.int16(-32768))
    bias = jnp.int16(-32768)

    def cond2(state):
        i, _, done = state
        return (i < 16) & (jnp.max(done) >= 0)

    def body2(state):
        i, p, done = state
        maskb = (jnp.int32(1) << (15 - i)).astype(jnp.int16)
        probe = (p | maskb) ^ bias
        cnt = _count_tree((w >= probe).astype(jnp.int16))
        s = ((cnt - r) >> 31).astype(jnp.int16)
        d = cnt ^ r
        eq = ~((d | (0 - d)) >> 31)
        p = p | (maskb & ~s & ~done.astype(jnp.int16))
        done = done | eq
        return i + 1, p, done

    _, L, _ = jax.lax.while_loop(
        cond2, body2,
        (jnp.int32(0), jnp.zeros((1, cols), jnp.int16),
         jnp.zeros((1, cols), jnp.int32)))

    thresh = (T.astype(jnp.int32) << 16) | (L.astype(jnp.int32) & 0xFFFF)
    z = jnp.where(bits >= thresh, acts, 0.0)
    out_ref[...] = jax.lax.dot_general(
        z, wdec_ref[...], (((0,), (0,)), ((), ())),
        preferred_element_type=jnp.float32) + b_dec_ref[...].T


@jax.jit
def kernel(x, W_enc, b_enc, W_dec, b_dec):
    batch, d_in = x.shape
    hidden = W_enc.shape[0]
    grid = (batch // _BLOCK_ROWS,)
    return pl.pallas_call(
        _sae_block_kernel,
        grid=grid,
        in_specs=[
            pl.BlockSpec((d_in, _BLOCK_ROWS), lambda i: (0, i)),
            pl.BlockSpec((hidden, d_in), lambda i: (0, 0)),
            pl.BlockSpec((hidden, 1), lambda i: (0, 0)),
            pl.BlockSpec((hidden, d_in), lambda i: (0, 0)),
            pl.BlockSpec((d_in, 1), lambda i: (0, 0)),
        ],
        out_specs=pl.BlockSpec((_BLOCK_ROWS, d_in), lambda i: (i, 0)),
        out_shape=jax.ShapeDtypeStruct((batch, d_in), jnp.float32),
    )(x.T, W_enc, b_enc.reshape(hidden, 1), W_dec, b_dec.reshape(d_in, 1))


# speedup vs baseline: 1.1885x; 1.1885x over previous
import functools

import jax
import jax.numpy as jnp
from jax.experimental import pallas as pl

_K = 32
_BLOCK_ROWS = 256


def _count_tree(msk01):
    h = msk01.shape[0]
    while h > 32:
        q = h // 4
        msk01 = ((msk01[:q, :] + msk01[q:2 * q, :])
                 + (msk01[2 * q:3 * q, :] + msk01[3 * q:, :]))
        h = q
    return jnp.sum(msk01.astype(jnp.int32), axis=0, keepdims=True)


def _sae_block_kernel(xt_ref, wenc_ref, b_enc_ref, wdec_ref, b_dec_ref,
                      out_ref):
    sae_in_t = xt_ref[...] - b_dec_ref[...]
    pre = jnp.dot(wenc_ref[...], sae_in_t,
                  preferred_element_type=jnp.float32)
    acts = jnp.maximum(pre + b_enc_ref[...], 0.0)
    bits = jax.lax.bitcast_convert_type(acts, jnp.int32)

    hi16 = (bits >> 16).astype(jnp.int16)
    lo16 = (bits ^ 0x8000).astype(jnp.int16)

    cols = acts.shape[1]

    def body1(i, p):
        maskb = (jnp.int32(1) << (14 - i)).astype(jnp.int16)
        probe = p | maskb
        cnt = _count_tree((hi16 >= probe).astype(jnp.int16))
        s = ((cnt - _K) >> 31).astype(jnp.int16)
        return p | (maskb & ~s)

    T = jax.lax.fori_loop(0, 15, body1, jnp.zeros((1, cols), jnp.int16))

    c_hi = _count_tree((hi16 > T).astype(jnp.int16))
    r = _K - c_hi

    w = jnp.where(hi16 == T, lo16, jnp.int16(-32768))
    bias = jnp.int16(-32768)

    def body2(i, p):
        maskb = (jnp.int32(1) << (15 - i)).astype(jnp.int16)
        probe = (p | maskb) ^ bias
        cnt = _count_tree((w >= probe).astype(jnp.int16))
        s = ((cnt - r) >> 31).astype(jnp.int16)
        return p | (maskb & ~s)

    L = jax.lax.fori_loop(0, 16, body2, jnp.zeros((1, cols), jnp.int16))

    thresh = (T.astype(jnp.int32) << 16) | (L.astype(jnp.int32) & 0xFFFF)
    z = jnp.where(bits >= thresh, acts, 0.0)
    out_ref[...] = jax.lax.dot_general(
        z, wdec_ref[...], (((0,), (0,)), ((), ())),
        preferred_element_type=jnp.float32) + b_dec_ref[...].T


@jax.jit
def kernel(x, W_enc, b_enc, W_dec, b_dec):
    batch, d_in = x.shape
    hidden = W_enc.shape[0]
    grid = (batch // _BLOCK_ROWS,)
    return pl.pallas_call(
        _sae_block_kernel,
        grid=grid,
        in_specs=[
            pl.BlockSpec((d_in, _BLOCK_ROWS), lambda i: (0, i)),
            pl.BlockSpec((hidden, d_in), lambda i: (0, 0)),
            pl.BlockSpec((hidden, 1), lambda i: (0, 0)),
            pl.BlockSpec((hidden, d_in), lambda i: (0, 0)),
            pl.BlockSpec((d_in, 1), lambda i: (0, 0)),
        ],
        out_specs=pl.BlockSpec((_BLOCK_ROWS, d_in), lambda i: (i, 0)),
        out_shape=jax.ShapeDtypeStruct((batch, d_in), jnp.float32),
    )(x.T, W_enc, b_enc.reshape(hidden, 1), W_dec, b_dec.reshape(d_in, 1))


# speedup vs baseline: 1.2250x; 1.0307x over previous
import functools

import jax
import jax.numpy as jnp
from jax.experimental import pallas as pl

_K = 32
_BLOCK_ROWS = 256


def _count_ge(arr, probe, strict=False):
    q = arr.shape[0] // 4
    op = (lambda a: (a > probe)) if strict else (lambda a: (a >= probe))
    m01 = ((op(arr[:q, :]).astype(jnp.int16)
            + op(arr[q:2 * q, :]).astype(jnp.int16))
           + (op(arr[2 * q:3 * q, :]).astype(jnp.int16)
              + op(arr[3 * q:, :]).astype(jnp.int16)))
    while m01.shape[0] > 32:
        q = m01.shape[0] // 4
        m01 = ((m01[:q, :] + m01[q:2 * q, :])
               + (m01[2 * q:3 * q, :] + m01[3 * q:, :]))
    return jnp.sum(m01.astype(jnp.int32), axis=0, keepdims=True)


def _sae_block_kernel(xt_ref, wenc_ref, b_enc_ref, wdec_ref, b_dec_ref,
                      out_ref):
    sae_in_t = xt_ref[...] - b_dec_ref[...]
    pre = jnp.dot(wenc_ref[...], sae_in_t,
                  preferred_element_type=jnp.float32)
    acts = jnp.maximum(pre + b_enc_ref[...], 0.0)
    bits = jax.lax.bitcast_convert_type(acts, jnp.int32)

    hi16 = (bits >> 16).astype(jnp.int16)
    lo16 = (bits ^ 0x8000).astype(jnp.int16)

    cols = acts.shape[1]

    T = jnp.zeros((1, cols), jnp.int16)
    for i in range(15):
        maskb = jnp.int16(1 << (14 - i))
        cnt = _count_ge(hi16, T | maskb)
        s = ((cnt - _K) >> 31).astype(jnp.int16)
        T = T | (maskb & ~s)

    c_hi = _count_ge(hi16, T, strict=True)
    r = _K - c_hi

    w = jnp.where(hi16 == T, lo16, jnp.int16(-32768))
    bias = jnp.int16(-32768)

    def body2(i, p):
        maskb = (jnp.int32(1) << (15 - i)).astype(jnp.int16)
        probe = (p | maskb) ^ bias
        cnt = _count_ge(w, probe)
        s = ((cnt - r) >> 31).astype(jnp.int16)
        return p | (maskb & ~s)

    L = jax.lax.fori_loop(0, 16, body2, jnp.zeros((1, cols), jnp.int16))

    thresh = (T.astype(jnp.int32) << 16) | (L.astype(jnp.int32) & 0xFFFF)
    z = jnp.where(bits >= thresh, acts, 0.0)
    out_ref[...] = jax.lax.dot_general(
        z, wdec_ref[...], (((0,), (0,)), ((), ())),
        preferred_element_type=jnp.float32) + b_dec_ref[...].T


@jax.jit
def kernel(x, W_enc, b_enc, W_dec, b_dec):
    batch, d_in = x.shape
    hidden = W_enc.shape[0]
    grid = (batch // _BLOCK_ROWS,)
    return pl.pallas_call(
        _sae_block_kernel,
        grid=grid,
        in_specs=[
            pl.BlockSpec((d_in, _BLOCK_ROWS), lambda i: (0, i)),
            pl.BlockSpec((hidden, d_in), lambda i: (0, 0)),
            pl.BlockSpec((hidden, 1), lambda i: (0, 0)),
            pl.BlockSpec((hidden, d_in), lambda i: (0, 0)),
            pl.BlockSpec((d_in, 1), lambda i: (0, 0)),
        ],
        out_specs=pl.BlockSpec((_BLOCK_ROWS, d_in), lambda i: (i, 0)),
        out_shape=jax.ShapeDtypeStruct((batch, d_in), jnp.float32),
    )(x.T, W_enc, b_enc.reshape(hidden, 1), W_dec, b_dec.reshape(d_in, 1))


# speedup vs baseline: 1.3029x; 1.0636x over previous
import functools

import jax
import jax.numpy as jnp
from jax.experimental import pallas as pl

_K = 32
_BLOCK_ROWS = 512


def _count_ge(arr, probe, strict=False):
    q = arr.shape[0] // 4
    op = (lambda a: (a > probe)) if strict else (lambda a: (a >= probe))
    m01 = ((op(arr[:q, :]).astype(jnp.int16)
            + op(arr[q:2 * q, :]).astype(jnp.int16))
           + (op(arr[2 * q:3 * q, :]).astype(jnp.int16)
              + op(arr[3 * q:, :]).astype(jnp.int16)))
    while m01.shape[0] > 32:
        q = m01.shape[0] // 4
        m01 = ((m01[:q, :] + m01[q:2 * q, :])
               + (m01[2 * q:3 * q, :] + m01[3 * q:, :]))
    return jnp.sum(m01.astype(jnp.int32), axis=0, keepdims=True)


def _sae_block_kernel(xt_ref, wenc_ref, b_enc_ref, wdec_ref, b_dec_ref,
                      out_ref):
    sae_in_t = xt_ref[...] - b_dec_ref[...]
    pre = jnp.dot(wenc_ref[...], sae_in_t,
                  preferred_element_type=jnp.float32)
    acts = jnp.maximum(pre + b_enc_ref[...], 0.0)
    bits = jax.lax.bitcast_convert_type(acts, jnp.int32)

    hi16 = (bits >> 16).astype(jnp.int16)
    lo16 = (bits ^ 0x8000).astype(jnp.int16)

    cols = acts.shape[1]

    T = jnp.zeros((1, cols), jnp.int16)
    for i in range(15):
        maskb = jnp.int16(1 << (14 - i))
        cnt = _count_ge(hi16, T | maskb)
        s = ((cnt - _K) >> 31).astype(jnp.int16)
        T = T | (maskb & ~s)

    c_hi = _count_ge(hi16, T, strict=True)
    r = _K - c_hi

    w = jnp.where(hi16 == T, lo16, jnp.int16(-32768))
    bias = jnp.int16(-32768)

    def body2(i, p):
        maskb = (jnp.int32(1) << (15 - i)).astype(jnp.int16)
        probe = (p | maskb) ^ bias
        cnt = _count_ge(w, probe)
        s = ((cnt - r) >> 31).astype(jnp.int16)
        return p | (maskb & ~s)

    L = jax.lax.fori_loop(0, 16, body2, jnp.zeros((1, cols), jnp.int16))

    thresh = (T.astype(jnp.int32) << 16) | (L.astype(jnp.int32) & 0xFFFF)
    z = jnp.where(bits >= thresh, acts, 0.0)
    out_ref[...] = jax.lax.dot_general(
        z, wdec_ref[...], (((0,), (0,)), ((), ())),
        preferred_element_type=jnp.float32) + b_dec_ref[...].T


@jax.jit
def kernel(x, W_enc, b_enc, W_dec, b_dec):
    batch, d_in = x.shape
    hidden = W_enc.shape[0]
    grid = (batch // _BLOCK_ROWS,)
    return pl.pallas_call(
        _sae_block_kernel,
        grid=grid,
        in_specs=[
            pl.BlockSpec((d_in, _BLOCK_ROWS), lambda i: (0, i)),
            pl.BlockSpec((hidden, d_in), lambda i: (0, 0)),
            pl.BlockSpec((hidden, 1), lambda i: (0, 0)),
            pl.BlockSpec((hidden, d_in), lambda i: (0, 0)),
            pl.BlockSpec((d_in, 1), lambda i: (0, 0)),
        ],
        out_specs=pl.BlockSpec((_BLOCK_ROWS, d_in), lambda i: (i, 0)),
        out_shape=jax.ShapeDtypeStruct((batch, d_in), jnp.float32),
    )(x.T, W_enc, b_enc.reshape(hidden, 1), W_dec, b_dec.reshape(d_in, 1))


# speedup vs baseline: 1.3317x; 1.0221x over previous
import functools

import jax
import jax.numpy as jnp
from jax.experimental import pallas as pl

_K = 32
_BLOCK_ROWS = 1024


def _count_ge(arr, probe, strict=False):
    q = arr.shape[0] // 4
    op = (lambda a: (a > probe)) if strict else (lambda a: (a >= probe))
    m01 = ((op(arr[:q, :]).astype(jnp.int16)
            + op(arr[q:2 * q, :]).astype(jnp.int16))
           + (op(arr[2 * q:3 * q, :]).astype(jnp.int16)
              + op(arr[3 * q:, :]).astype(jnp.int16)))
    while m01.shape[0] > 32:
        q = m01.shape[0] // 4
        m01 = ((m01[:q, :] + m01[q:2 * q, :])
               + (m01[2 * q:3 * q, :] + m01[3 * q:, :]))
    return jnp.sum(m01.astype(jnp.int32), axis=0, keepdims=True)


def _sae_block_kernel(xt_ref, wenc_ref, b_enc_ref, wdec_ref, b_dec_ref,
                      out_ref):
    sae_in_t = xt_ref[...] - b_dec_ref[...]
    pre = jnp.dot(wenc_ref[...], sae_in_t,
                  preferred_element_type=jnp.float32)
    acts = jnp.maximum(pre + b_enc_ref[...], 0.0)
    bits = jax.lax.bitcast_convert_type(acts, jnp.int32)

    hi16 = (bits >> 16).astype(jnp.int16)
    lo16 = (bits ^ 0x8000).astype(jnp.int16)

    cols = acts.shape[1]

    T = jnp.zeros((1, cols), jnp.int16)
    for i in range(15):
        maskb = jnp.int16(1 << (14 - i))
        cnt = _count_ge(hi16, T | maskb)
        s = ((cnt - _K) >> 31).astype(jnp.int16)
        T = T | (maskb & ~s)

    c_hi = _count_ge(hi16, T, strict=True)
    r = _K - c_hi

    w = jnp.where(hi16 == T, lo16, jnp.int16(-32768))
    bias = jnp.int16(-32768)

    def body2(i, p):
        maskb = (jnp.int32(1) << (15 - i)).astype(jnp.int16)
        probe = (p | maskb) ^ bias
        cnt = _count_ge(w, probe)
        s = ((cnt - r) >> 31).astype(jnp.int16)
        return p | (maskb & ~s)

    L = jax.lax.fori_loop(0, 16, body2, jnp.zeros((1, cols), jnp.int16))

    thresh = (T.astype(jnp.int32) << 16) | (L.astype(jnp.int32) & 0xFFFF)
    z = jnp.where(bits >= thresh, acts, 0.0)
    out_ref[...] = jax.lax.dot_general(
        z, wdec_ref[...], (((0,), (0,)), ((), ())),
        preferred_element_type=jnp.float32) + b_dec_ref[...].T


@jax.jit
def kernel(x, W_enc, b_enc, W_dec, b_dec):
    batch, d_in = x.shape
    hidden = W_enc.shape[0]
    grid = (batch // _BLOCK_ROWS,)
    return pl.pallas_call(
        _sae_block_kernel,
        grid=grid,
        in_specs=[
            pl.BlockSpec((d_in, _BLOCK_ROWS), lambda i: (0, i)),
            pl.BlockSpec((hidden, d_in), lambda i: (0, 0)),
            pl.BlockSpec((hidden, 1), lambda i: (0, 0)),
            pl.BlockSpec((hidden, d_in), lambda i: (0, 0)),
            pl.BlockSpec((d_in, 1), lambda i: (0, 0)),
        ],
        out_specs=pl.BlockSpec((_BLOCK_ROWS, d_in), lambda i: (i, 0)),
        out_shape=jax.ShapeDtypeStruct((batch, d_in), jnp.float32),
    )(x.T, W_enc, b_enc.reshape(hidden, 1), W_dec, b_dec.reshape(d_in, 1))
